# baseline (speedup 1.0000x reference)
import functools

import jax
import jax.numpy as jnp
from jax import lax
from jax.experimental import pallas as pl
from jax.experimental.pallas import tpu as pltpu
from jax.experimental.pallas import tpu_sc as plsc

_N = 10000
_E = 160000
_D = 256
_NL = 4
_NS = 16
_NC = 2

_HALF = 5120
_CB = 64
_IW = 2 * _CB
_CH = 88
_IG = 8
_SLOT = _NS * _CH * _CB
_AR = 10496
_ZT = _AR // _NS
_NPG = 10368

_DCB = 128
_DCH = 80
_DEPAD = _NS * _DCH * _DCB
_ACCR = 10240
_ZR = _ACCR // _NS

_mesh = plsc.VectorSubcoreMesh(core_axis_name="c", subcore_axis_name="s")
_f32 = jnp.float32


@functools.partial(
    pl.kernel,
    out_type=jax.ShapeDtypeStruct((_NC, _ACCR, 128), _f32),
    mesh=_mesh,
    scratch_types=[
        pltpu.VMEM((_DCH, _DCB), jnp.int32),
        pltpu.VMEM((_DCB, 128), _f32),
        pltpu.MemorySpace.VMEM_SHARED((_ACCR, 128), _f32),
    ],
)
def _deg_pass(dst_hbm, ones_hbm, zer_hbm, out_hbm, dst_v, ones_v, acc):
    c = lax.axis_index("c")
    s = lax.axis_index("s")
    pltpu.sync_copy(dst_hbm.at[s], dst_v)
    pltpu.sync_copy(ones_hbm, ones_v)
    pltpu.sync_copy(zer_hbm, acc.at[pl.ds(s * _ZR, _ZR)])
    plsc.subcore_barrier()

    def body(j, carry):
        pltpu.sync_copy(ones_v, acc.at[dst_v.at[j]], add=True)
        return carry

    lax.fori_loop(0, _DCH, body, 0)
    plsc.subcore_barrier()
    pltpu.sync_copy(acc.at[pl.ds(s * _ZR, _ZR)],
                    out_hbm.at[c].at[pl.ds(s * _ZR, _ZR)])


@functools.partial(
    pl.kernel,
    out_type=jax.ShapeDtypeStruct((_NC, _AR, 128), _f32),
    mesh=_mesh,
    scratch_types=[
        pltpu.VMEM((_IG, _IW), jnp.int32),
        pltpu.VMEM((_IG, _IW), jnp.int32),
        pltpu.VMEM((_IW, 128), _f32),
        pltpu.VMEM((_IW, 128), _f32),
        pltpu.MemorySpace.VMEM_SHARED((_AR, 128), _f32),
        pltpu.SemaphoreType.DMA,
        pltpu.SemaphoreType.DMA,
    ],
)
def _edge_pass(g_hbm, src_hbm, dst_hbm, out_hbm,
               src_v, dst_v, buf0, buf1, acc, sem0, sem1):
    c = lax.axis_index("c")
    s = lax.axis_index("s")
    pltpu.sync_copy(g_hbm.at[pl.ds(2 * c * _HALF + s * _ZT, _ZT)],
                    acc.at[pl.ds(s * _ZT, _ZT)])
    plsc.subcore_barrier()

    sh = src_hbm.at[c].at[s]
    dh = dst_hbm.at[c].at[s]
    for grp in range(_CH // _IG):
        pltpu.sync_copy(sh.at[pl.ds(grp * _IG, _IG)], src_v)
        pltpu.sync_copy(dh.at[pl.ds(grp * _IG, _IG)], dst_v)
        pltpu.async_copy(g_hbm.at[src_v.at[0]], buf0, sem0)

        def body(it, carry):
            j = it * 2
            pltpu.async_copy(g_hbm.at[src_v.at[j + 1]], buf1, sem1)
            pltpu.make_async_copy(g_hbm.at[src_v.at[j]], buf0, sem0).wait()
            pltpu.sync_copy(buf0, acc.at[dst_v.at[j]], add=True)

            @pl.when(it < _IG // 2 - 1)
            def _():
                pltpu.async_copy(g_hbm.at[src_v.at[j + 2]], buf0, sem0)

            pltpu.make_async_copy(g_hbm.at[src_v.at[j + 1]], buf1,
                                  sem1).wait()
            pltpu.sync_copy(buf1, acc.at[dst_v.at[j + 1]], add=True)
            return carry

        lax.fori_loop(0, _IG // 2, body, 0)
    plsc.subcore_barrier()
    pltpu.sync_copy(acc.at[pl.ds(s * _ZT, _ZT)],
                    out_hbm.at[c].at[pl.ds(s * _ZT, _ZT)])


def _dinv_from(deg_ref):
    deg = deg_ref[0:_N, 0:1] + 1.0
    return lax.rsqrt(jnp.maximum(deg, 1.0))


def _bn_core(h_ref, s_ref, dinv, b_ref, gm_ref, bt_ref):
    s_full = jnp.concatenate(
        [s_ref[0, 0:_HALF, :], s_ref[1, 0:_N - _HALF, :]], axis=0)
    agg = dinv * s_full + b_ref[...]
    mu = jnp.mean(agg, axis=0, keepdims=True)
    xc = agg - mu
    var = jnp.mean(xc * xc, axis=0, keepdims=True)
    hbn = xc * lax.rsqrt(var + 1e-5) * gm_ref[...] + bt_ref[...]
    return h_ref[...] + jnp.maximum(hbn, 0.0)


def _emb_body(x_ref, we_ref, be_ref, h_ref):
    h = jnp.dot(x_ref[...], we_ref[...], preferred_element_type=_f32)
    h_ref[...] = h + be_ref[...]


def _mm_body(h_ref, w_ref, deg_ref, g_ref):
    dinv = _dinv_from(deg_ref)
    ht = jnp.dot(h_ref[...], w_ref[...], preferred_element_type=_f32)
    g_ref[0:_N, :] = dinv * ht
    g_ref[_N:_NPG, :] = jnp.zeros((_NPG - _N, _D), _f32)


def _bn_body(h_ref, s_ref, deg_ref, b_ref, gm_ref, bt_ref, hn_ref):
    dinv = _dinv_from(deg_ref)
    hn_ref[...] = _bn_core(h_ref, s_ref, dinv, b_ref, gm_ref, bt_ref)


def _fin_body(h_ref, s_ref, deg_ref, b_ref, gm_ref, bt_ref,
              w1_ref, b1_ref, w2_ref, b2_ref, w3_ref, b3_ref, out_ref):
    dinv = _dinv_from(deg_ref)
    hn = _bn_core(h_ref, s_ref, dinv, b_ref, gm_ref, bt_ref)
    r = jnp.dot(hn, w1_ref[...], preferred_element_type=_f32) + b1_ref[...]
    r = jnp.maximum(r, 0.0)
    r = jnp.dot(r, w2_ref[...], preferred_element_type=_f32) + b2_ref[...]
    r = jnp.maximum(r, 0.0)
    out_ref[...] = jnp.dot(r, w3_ref[...], preferred_element_type=_f32) + b3_ref[...]


_sds = jax.ShapeDtypeStruct

_emb_call = pl.pallas_call(_emb_body, out_shape=_sds((_N, _D), _f32))
_mm_call = pl.pallas_call(_mm_body, out_shape=_sds((_NPG, _D), _f32))
_bn_call = pl.pallas_call(_bn_body, out_shape=_sds((_N, _D), _f32))
_fin_call = pl.pallas_call(_fin_body, out_shape=_sds((_N, 6), _f32))


def kernel(x, edge_index, We, be, convW, convb, gamma, beta,
           W1, b1, W2, b2, W3, b3):
    src = edge_index[0].astype(jnp.int32)
    dst = edge_index[1].astype(jnp.int32)

    key = (dst >= _HALF).astype(jnp.int32)
    c0 = _E - jnp.sum(key)
    dstl = jnp.where(key == 0, dst, dst - _HALF)
    _, ssrc, sdst = lax.sort((key, src, dstl), dimension=0, num_keys=1,
                             is_stable=True)
    start = _SLOT - c0
    slot_iota = jnp.arange(2 * _SLOT, dtype=jnp.int32)
    src_bg = _N + slot_iota % (_NPG - _N)
    dst_bg = _HALF + 1 + slot_iota % (_AR // 2 - _HALF - 1)
    srcf = lax.dynamic_update_slice(src_bg, ssrc, (start,))
    dstf = lax.dynamic_update_slice(dst_bg, sdst, (start,))

    def _ileave(v):
        return jnp.stack([2 * v, 2 * v + 1],
                         axis=-1).reshape(_NC, _NS, _CH, _IW)

    srcp = _ileave(srcf)
    dstp = _ileave(dstf)

    dpad = jnp.full((_DEPAD - _E,), _N, jnp.int32)
    dstp_deg = jnp.concatenate([dst, dpad]).reshape(_NS, _DCH, _DCB)
    ones_b = jnp.ones((_DCB, 128), _f32)
    zer_b = jnp.zeros((_ZR, 128), _f32)
    deg2d = _deg_pass(dstp_deg, ones_b, zer_b)[0]

    be2 = be.reshape(1, _D)
    h = _emb_call(x, We, be2)
    for i in range(_NL - 1):
        g = _mm_call(h, convW[i], deg2d)
        s_agg = _edge_pass(g.reshape(2 * _NPG, 128), srcp, dstp)
        s_agg = s_agg.reshape(_NC, _AR // 2, _D)
        h = _bn_call(h, s_agg, deg2d, convb[i].reshape(1, _D),
                     gamma[i].reshape(1, _D), beta[i].reshape(1, _D))
    i = _NL - 1
    g = _mm_call(h, convW[i], deg2d)
    s_agg = _edge_pass(g.reshape(2 * _NPG, 128), srcp, dstp)
    s_agg = s_agg.reshape(_NC, _AR // 2, _D)
    return _fin_call(h, s_agg, deg2d, convb[i].reshape(1, _D),
                     gamma[i].reshape(1, _D), beta[i].reshape(1, _D),
                     W1, b1.reshape(1, -1), W2, b2.reshape(1, -1),
                     W3, b3.reshape(1, -1))


# speedup vs baseline: 1.0171x; 1.0171x over previous
import functools

import jax
import jax.numpy as jnp
from jax import lax
from jax.experimental import pallas as pl
from jax.experimental.pallas import tpu as pltpu
from jax.experimental.pallas import tpu_sc as plsc

_N = 10000
_E = 160000
_D = 256
_NL = 4
_NS = 16
_NC = 2

_HALF = 5120
_CB = 64
_IW = 2 * _CB
_CH = 96
_IG = 24
_SLOT = _NS * _CH * _CB
_AR = 10496
_ZT = _AR // _NS
_NPG = 10368

_DCB = 128
_DCH = 80
_DEPAD = _NS * _DCH * _DCB
_ACCR = 10240
_ZR = _ACCR // _NS

_mesh = plsc.VectorSubcoreMesh(core_axis_name="c", subcore_axis_name="s")
_f32 = jnp.float32


@functools.partial(
    pl.kernel,
    out_type=jax.ShapeDtypeStruct((_NC, _ACCR, 128), _f32),
    mesh=_mesh,
    scratch_types=[
        pltpu.VMEM((_DCH, _DCB), jnp.int32),
        pltpu.VMEM((_DCB, 128), _f32),
        pltpu.MemorySpace.VMEM_SHARED((_ACCR, 128), _f32),
    ],
)
def _deg_pass(dst_hbm, ones_hbm, zer_hbm, out_hbm, dst_v, ones_v, acc):
    c = lax.axis_index("c")
    s = lax.axis_index("s")
    pltpu.sync_copy(dst_hbm.at[s], dst_v)
    pltpu.sync_copy(ones_hbm, ones_v)
    pltpu.sync_copy(zer_hbm, acc.at[pl.ds(s * _ZR, _ZR)])
    plsc.subcore_barrier()

    def body(j, carry):
        pltpu.sync_copy(ones_v, acc.at[dst_v.at[j]], add=True)
        return carry

    lax.fori_loop(0, _DCH, body, 0)
    plsc.subcore_barrier()
    pltpu.sync_copy(acc.at[pl.ds(s * _ZR, _ZR)],
                    out_hbm.at[c].at[pl.ds(s * _ZR, _ZR)])


@functools.partial(
    pl.kernel,
    out_type=jax.ShapeDtypeStruct((_NC, _AR, 128), _f32),
    mesh=_mesh,
    scratch_types=[
        pltpu.VMEM((_IG, _IW), jnp.int32),
        pltpu.VMEM((_IG, _IW), jnp.int32),
        pltpu.VMEM((_IW, 128), _f32),
        pltpu.VMEM((_IW, 128), _f32),
        pltpu.MemorySpace.VMEM_SHARED((_AR, 128), _f32),
        pltpu.SemaphoreType.DMA,
        pltpu.SemaphoreType.DMA,
    ],
)
def _edge_pass(g_hbm, src_hbm, dst_hbm, out_hbm,
               src_v, dst_v, buf0, buf1, acc, sem0, sem1):
    c = lax.axis_index("c")
    s = lax.axis_index("s")
    pltpu.sync_copy(g_hbm.at[pl.ds(2 * c * _HALF + s * _ZT, _ZT)],
                    acc.at[pl.ds(s * _ZT, _ZT)])
    plsc.subcore_barrier()

    sh = src_hbm.at[c].at[s]
    dh = dst_hbm.at[c].at[s]
    for grp in range(_CH // _IG):
        pltpu.sync_copy(sh.at[pl.ds(grp * _IG, _IG)], src_v)
        pltpu.sync_copy(dh.at[pl.ds(grp * _IG, _IG)], dst_v)
        pltpu.async_copy(g_hbm.at[src_v.at[0]], buf0, sem0)

        def body(it, carry):
            j = it * 2
            pltpu.async_copy(g_hbm.at[src_v.at[j + 1]], buf1, sem1)
            pltpu.make_async_copy(g_hbm.at[src_v.at[j]], buf0, sem0).wait()
            pltpu.sync_copy(buf0, acc.at[dst_v.at[j]], add=True)

            @pl.when(it < _IG // 2 - 1)
            def _():
                pltpu.async_copy(g_hbm.at[src_v.at[j + 2]], buf0, sem0)

            pltpu.make_async_copy(g_hbm.at[src_v.at[j + 1]], buf1,
                                  sem1).wait()
            pltpu.sync_copy(buf1, acc.at[dst_v.at[j + 1]], add=True)
            return carry

        lax.fori_loop(0, _IG // 2, body, 0)
    plsc.subcore_barrier()
    pltpu.sync_copy(acc.at[pl.ds(s * _ZT, _ZT)],
                    out_hbm.at[c].at[pl.ds(s * _ZT, _ZT)])


def _dinv_from(deg_ref):
    deg = deg_ref[0:_N, 0:1] + 1.0
    return lax.rsqrt(jnp.maximum(deg, 1.0))


def _bn_core(h_ref, s_ref, dinv, b_ref, gm_ref, bt_ref):
    s_full = jnp.concatenate(
        [s_ref[0, 0:_HALF, :], s_ref[1, 0:_N - _HALF, :]], axis=0)
    agg = dinv * s_full + b_ref[...]
    mu = jnp.mean(agg, axis=0, keepdims=True)
    xc = agg - mu
    var = jnp.mean(xc * xc, axis=0, keepdims=True)
    hbn = xc * lax.rsqrt(var + 1e-5) * gm_ref[...] + bt_ref[...]
    return h_ref[...] + jnp.maximum(hbn, 0.0)


def _emb_body(x_ref, we_ref, be_ref, h_ref):
    h = jnp.dot(x_ref[...], we_ref[...], preferred_element_type=_f32)
    h_ref[...] = h + be_ref[...]


def _mm_body(h_ref, w_ref, deg_ref, g_ref):
    dinv = _dinv_from(deg_ref)
    ht = jnp.dot(h_ref[...], w_ref[...], preferred_element_type=_f32)
    g_ref[0:_N, :] = dinv * ht
    g_ref[_N:_NPG, :] = jnp.zeros((_NPG - _N, _D), _f32)


def _bn_body(h_ref, s_ref, deg_ref, b_ref, gm_ref, bt_ref, hn_ref):
    dinv = _dinv_from(deg_ref)
    hn_ref[...] = _bn_core(h_ref, s_ref, dinv, b_ref, gm_ref, bt_ref)


def _fin_body(h_ref, s_ref, deg_ref, b_ref, gm_ref, bt_ref,
              w1_ref, b1_ref, w2_ref, b2_ref, w3_ref, b3_ref, out_ref):
    dinv = _dinv_from(deg_ref)
    hn = _bn_core(h_ref, s_ref, dinv, b_ref, gm_ref, bt_ref)
    r = jnp.dot(hn, w1_ref[...], preferred_element_type=_f32) + b1_ref[...]
    r = jnp.maximum(r, 0.0)
    r = jnp.dot(r, w2_ref[...], preferred_element_type=_f32) + b2_ref[...]
    r = jnp.maximum(r, 0.0)
    out_ref[...] = jnp.dot(r, w3_ref[...], preferred_element_type=_f32) + b3_ref[...]


_sds = jax.ShapeDtypeStruct

_emb_call = pl.pallas_call(_emb_body, out_shape=_sds((_N, _D), _f32))
_mm_call = pl.pallas_call(_mm_body, out_shape=_sds((_NPG, _D), _f32))
_bn_call = pl.pallas_call(_bn_body, out_shape=_sds((_N, _D), _f32))
_fin_call = pl.pallas_call(_fin_body, out_shape=_sds((_N, 6), _f32))


def kernel(x, edge_index, We, be, convW, convb, gamma, beta,
           W1, b1, W2, b2, W3, b3):
    src = edge_index[0].astype(jnp.int32)
    dst = edge_index[1].astype(jnp.int32)

    key = (dst >= _HALF).astype(jnp.int32)
    c0 = _E - jnp.sum(key)
    dstl = jnp.where(key == 0, dst, dst - _HALF)
    _, ssrc, sdst = lax.sort((key, src, dstl), dimension=0, num_keys=1,
                             is_stable=True)
    start = _SLOT - c0
    slot_iota = jnp.arange(2 * _SLOT, dtype=jnp.int32)
    src_bg = _N + slot_iota % (_NPG - _N)
    dst_bg = _HALF + 1 + slot_iota % (_AR // 2 - _HALF - 1)
    srcf = lax.dynamic_update_slice(src_bg, ssrc, (start,))
    dstf = lax.dynamic_update_slice(dst_bg, sdst, (start,))

    def _ileave(v):
        return jnp.stack([2 * v, 2 * v + 1],
                         axis=-1).reshape(_NC, _NS, _CH, _IW)

    srcp = _ileave(srcf)
    dstp = _ileave(dstf)

    dpad = jnp.full((_DEPAD - _E,), _N, jnp.int32)
    dstp_deg = jnp.concatenate([dst, dpad]).reshape(_NS, _DCH, _DCB)
    ones_b = jnp.ones((_DCB, 128), _f32)
    zer_b = jnp.zeros((_ZR, 128), _f32)
    deg2d = _deg_pass(dstp_deg, ones_b, zer_b)[0]

    be2 = be.reshape(1, _D)
    h = _emb_call(x, We, be2)
    for i in range(_NL - 1):
        g = _mm_call(h, convW[i], deg2d)
        s_agg = _edge_pass(g.reshape(2 * _NPG, 128), srcp, dstp)
        s_agg = s_agg.reshape(_NC, _AR // 2, _D)
        h = _bn_call(h, s_agg, deg2d, convb[i].reshape(1, _D),
                     gamma[i].reshape(1, _D), beta[i].reshape(1, _D))
    i = _NL - 1
    g = _mm_call(h, convW[i], deg2d)
    s_agg = _edge_pass(g.reshape(2 * _NPG, 128), srcp, dstp)
    s_agg = s_agg.reshape(_NC, _AR // 2, _D)
    return _fin_call(h, s_agg, deg2d, convb[i].reshape(1, _D),
                     gamma[i].reshape(1, _D), beta[i].reshape(1, _D),
                     W1, b1.reshape(1, -1), W2, b2.reshape(1, -1),
                     W3, b3.reshape(1, -1))


# speedup vs baseline: 1.0592x; 1.0414x over previous
import functools

import jax
import jax.numpy as jnp
from jax import lax
from jax.experimental import pallas as pl
from jax.experimental.pallas import tpu as pltpu
from jax.experimental.pallas import tpu_sc as plsc

_N = 10000
_E = 160000
_D = 256
_NL = 4
_NS = 16
_NC = 2

_HALF = 5120
_CB = 64
_IW = 2 * _CB
_CH = 96
_IG = 24
_SLOT = _NS * _CH * _CB
_AR = 10496
_ZT = _AR // _NS
_NPG = 10368

_DCB = 128
_DCH = 80
_DEPAD = _NS * _DCH * _DCB
_ACCR = 10240
_ZR = _ACCR // _NS

_mesh = plsc.VectorSubcoreMesh(core_axis_name="c", subcore_axis_name="s")
_f32 = jnp.float32


@functools.partial(
    pl.kernel,
    out_type=jax.ShapeDtypeStruct((_NC, _ACCR, 128), _f32),
    mesh=_mesh,
    scratch_types=[
        pltpu.VMEM((_DCH, _DCB), jnp.int32),
        pltpu.VMEM((_DCB, 128), _f32),
        pltpu.MemorySpace.VMEM_SHARED((_ACCR, 128), _f32),
    ],
)
def _deg_pass(dst_hbm, ones_hbm, zer_hbm, out_hbm, dst_v, ones_v, acc):
    c = lax.axis_index("c")
    s = lax.axis_index("s")
    pltpu.sync_copy(dst_hbm.at[s], dst_v)
    pltpu.sync_copy(ones_hbm, ones_v)
    pltpu.sync_copy(zer_hbm, acc.at[pl.ds(s * _ZR, _ZR)])
    plsc.subcore_barrier()

    def body(j, carry):
        pltpu.sync_copy(ones_v, acc.at[dst_v.at[j]], add=True)
        return carry

    lax.fori_loop(0, _DCH, body, 0)
    plsc.subcore_barrier()
    pltpu.sync_copy(acc.at[pl.ds(s * _ZR, _ZR)],
                    out_hbm.at[c].at[pl.ds(s * _ZR, _ZR)])


@functools.partial(
    pl.kernel,
    out_type=jax.ShapeDtypeStruct((_NC, _AR, 128), _f32),
    mesh=_mesh,
    scratch_types=[
        pltpu.VMEM((_IG, _IW), jnp.int32),
        pltpu.VMEM((_IG, _IW), jnp.int32),
        pltpu.VMEM((_IW, 128), _f32),
        pltpu.VMEM((_IW, 128), _f32),
        pltpu.MemorySpace.VMEM_SHARED((_AR, 128), _f32),
        pltpu.SemaphoreType.DMA,
        pltpu.SemaphoreType.DMA,
    ],
)
def _edge_pass(g_hbm, src_hbm, dst_hbm, out_hbm,
               src_v, dst_v, buf0, buf1, acc, sem0, sem1):
    c = lax.axis_index("c")
    s = lax.axis_index("s")
    pltpu.sync_copy(g_hbm.at[pl.ds(2 * c * _HALF + s * _ZT, _ZT)],
                    acc.at[pl.ds(s * _ZT, _ZT)])
    plsc.subcore_barrier()

    sh = src_hbm.at[c].at[s]
    dh = dst_hbm.at[c].at[s]
    for grp in range(_CH // _IG):
        pltpu.sync_copy(sh.at[pl.ds(grp * _IG, _IG)], src_v)
        pltpu.sync_copy(dh.at[pl.ds(grp * _IG, _IG)], dst_v)
        pltpu.async_copy(g_hbm.at[src_v.at[0]], buf0, sem0)

        def body(it, carry):
            j = it * 2
            pltpu.async_copy(g_hbm.at[src_v.at[j + 1]], buf1, sem1)
            pltpu.make_async_copy(g_hbm.at[src_v.at[j]], buf0, sem0).wait()
            pltpu.sync_copy(buf0, acc.at[dst_v.at[j]], add=True)

            @pl.when(it < _IG // 2 - 1)
            def _():
                pltpu.async_copy(g_hbm.at[src_v.at[j + 2]], buf0, sem0)

            pltpu.make_async_copy(g_hbm.at[src_v.at[j + 1]], buf1,
                                  sem1).wait()
            pltpu.sync_copy(buf1, acc.at[dst_v.at[j + 1]], add=True)
            return carry

        lax.fori_loop(0, _IG // 2, body, 0)
    plsc.subcore_barrier()
    pltpu.sync_copy(acc.at[pl.ds(s * _ZT, _ZT)],
                    out_hbm.at[c].at[pl.ds(s * _ZT, _ZT)])


def _dinv_from(deg_ref):
    deg = deg_ref[0:_N, 0:1] + 1.0
    return lax.rsqrt(jnp.maximum(deg, 1.0))


def _bn_core(h_ref, s_ref, dinv, b_ref, gm_ref, bt_ref):
    s_full = jnp.concatenate(
        [s_ref[0, 0:_HALF, :], s_ref[1, 0:_N - _HALF, :]], axis=0)
    agg = dinv * s_full + b_ref[...]
    mu = jnp.mean(agg, axis=0, keepdims=True)
    xc = agg - mu
    var = jnp.mean(xc * xc, axis=0, keepdims=True)
    hbn = xc * lax.rsqrt(var + 1e-5) * gm_ref[...] + bt_ref[...]
    return h_ref[...] + jnp.maximum(hbn, 0.0)


def _emb_body(x_ref, we_ref, be_ref, h_ref):
    h = jnp.dot(x_ref[...], we_ref[...], preferred_element_type=_f32)
    h_ref[...] = h + be_ref[...]


def _mm_body(h_ref, w_ref, deg_ref, g_ref):
    dinv = _dinv_from(deg_ref)
    ht = jnp.dot(h_ref[...], w_ref[...], preferred_element_type=_f32)
    g_ref[0:_N, :] = dinv * ht
    g_ref[_N:_NPG, :] = jnp.zeros((_NPG - _N, _D), _f32)


def _bn_body(h_ref, s_ref, deg_ref, b_ref, gm_ref, bt_ref, hn_ref):
    dinv = _dinv_from(deg_ref)
    hn_ref[...] = _bn_core(h_ref, s_ref, dinv, b_ref, gm_ref, bt_ref)


def _fin_body(h_ref, s_ref, deg_ref, b_ref, gm_ref, bt_ref,
              w1_ref, b1_ref, w2_ref, b2_ref, w3_ref, b3_ref, out_ref):
    dinv = _dinv_from(deg_ref)
    hn = _bn_core(h_ref, s_ref, dinv, b_ref, gm_ref, bt_ref)
    r = jnp.dot(hn, w1_ref[...], preferred_element_type=_f32) + b1_ref[...]
    r = jnp.maximum(r, 0.0)
    r = jnp.dot(r, w2_ref[...], preferred_element_type=_f32) + b2_ref[...]
    r = jnp.maximum(r, 0.0)
    out_ref[...] = jnp.dot(r, w3_ref[...], preferred_element_type=_f32) + b3_ref[...]


_sds = jax.ShapeDtypeStruct

_emb_call = pl.pallas_call(_emb_body, out_shape=_sds((_N, _D), _f32))
_mm_call = pl.pallas_call(_mm_body, out_shape=_sds((_NPG, _D), _f32))
_bn_call = pl.pallas_call(_bn_body, out_shape=_sds((_N, _D), _f32))
_fin_call = pl.pallas_call(_fin_body, out_shape=_sds((_N, 6), _f32))


def kernel(x, edge_index, We, be, convW, convb, gamma, beta,
           W1, b1, W2, b2, W3, b3):
    src = edge_index[0].astype(jnp.int32)
    dst = edge_index[1].astype(jnp.int32)

    key = (dst >= _HALF).astype(jnp.int32)
    c0 = _E - jnp.sum(key)
    order = jnp.argsort(key, stable=True)
    dstl = jnp.where(key == 0, dst, dst - _HALF)
    ssrc = src[order]
    sdst = dstl[order]
    start = _SLOT - c0
    slot_iota = jnp.arange(2 * _SLOT, dtype=jnp.int32)
    src_bg = _N + slot_iota % (_NPG - _N)
    dst_bg = _HALF + 1 + slot_iota % (_AR // 2 - _HALF - 1)
    srcf = lax.dynamic_update_slice(src_bg, ssrc, (start,))
    dstf = lax.dynamic_update_slice(dst_bg, sdst, (start,))

    def _ileave(v):
        return jnp.stack([2 * v, 2 * v + 1],
                         axis=-1).reshape(_NC, _NS, _CH, _IW)

    srcp = _ileave(srcf)
    dstp = _ileave(dstf)

    dpad = jnp.full((_DEPAD - _E,), _N, jnp.int32)
    dstp_deg = jnp.concatenate([dst, dpad]).reshape(_NS, _DCH, _DCB)
    ones_b = jnp.ones((_DCB, 128), _f32)
    zer_b = jnp.zeros((_ZR, 128), _f32)
    deg2d = _deg_pass(dstp_deg, ones_b, zer_b)[0]

    be2 = be.reshape(1, _D)
    h = _emb_call(x, We, be2)
    for i in range(_NL - 1):
        g = _mm_call(h, convW[i], deg2d)
        s_agg = _edge_pass(g.reshape(2 * _NPG, 128), srcp, dstp)
        s_agg = s_agg.reshape(_NC, _AR // 2, _D)
        h = _bn_call(h, s_agg, deg2d, convb[i].reshape(1, _D),
                     gamma[i].reshape(1, _D), beta[i].reshape(1, _D))
    i = _NL - 1
    g = _mm_call(h, convW[i], deg2d)
    s_agg = _edge_pass(g.reshape(2 * _NPG, 128), srcp, dstp)
    s_agg = s_agg.reshape(_NC, _AR // 2, _D)
    return _fin_call(h, s_agg, deg2d, convb[i].reshape(1, _D),
                     gamma[i].reshape(1, _D), beta[i].reshape(1, _D),
                     W1, b1.reshape(1, -1), W2, b2.reshape(1, -1),
                     W3, b3.reshape(1, -1))
